# baseline (speedup 1.0000x reference)
import functools

import jax
import jax.numpy as jnp
from jax import lax
from jax.experimental import pallas as pl
from jax.experimental.pallas import tpu as pltpu
from jax.experimental.pallas import tpu_sc as plsc

NC = 2
NS = 16
NW = NC * NS

B = 16384 * 26
D = 64
B_PER_W = B // NW
CHUNK = 832
N_CHUNKS = B_PER_W // CHUNK


def _gather_body(table_hbm, idx_hbm, out_hbm, idx_v, rows_v, sem):
    wid = lax.axis_index("s") * NC + lax.axis_index("c")
    base = wid * B_PER_W
    for chunk in range(N_CHUNKS):
        off = base + chunk * CHUNK
        pltpu.sync_copy(idx_hbm.at[pl.ds(off, CHUNK)], idx_v)
        pltpu.async_copy(table_hbm.at[idx_v], rows_v, sem).wait()
        pltpu.sync_copy(rows_v, out_hbm.at[pl.ds(off, CHUNK)])


_gather_call = functools.partial(
    pl.kernel,
    out_type=jax.ShapeDtypeStruct((B, D), jnp.float32),
    mesh=plsc.VectorSubcoreMesh(core_axis_name="c", subcore_axis_name="s"),
    scratch_types=[
        pltpu.VMEM((CHUNK,), jnp.int32),
        pltpu.VMEM((CHUNK, D), jnp.float32),
        pltpu.SemaphoreType.DMA,
    ],
    compiler_params=pltpu.CompilerParams(use_tc_tiling_on_sc=False),
)(_gather_body)


@jax.jit
def kernel(arg0_1, arg1_1):
    idx = arg0_1.reshape(-1).astype(jnp.int32)
    out = _gather_call(arg1_1, idx)
    return (out.reshape(arg0_1.shape + (D,)),)


# speedup vs baseline: 1.0165x; 1.0165x over previous
import functools

import jax
import jax.numpy as jnp
from jax import lax
from jax.experimental import pallas as pl
from jax.experimental.pallas import tpu as pltpu
from jax.experimental.pallas import tpu_sc as plsc

NC = 2
NS = 16
NW = NC * NS

B = 16384 * 26
D = 64
B_PER_W = B // NW
CHUNK = 512
N_CHUNKS = B_PER_W // CHUNK
NBUF = 3


def _gather_body(table_hbm, idx_hbm, out_hbm, idx_all, *rest):
    rows = rest[:NBUF]
    gsems = rest[NBUF:2 * NBUF]
    wsems = rest[2 * NBUF:3 * NBUF]
    wid = lax.axis_index("s") * NC + lax.axis_index("c")
    base = wid * B_PER_W
    pltpu.sync_copy(idx_hbm.at[wid], idx_all)

    hg = {}
    hw = {}

    def issue(i):
        b = i % NBUF
        if i >= NBUF:
            hw[i - NBUF].wait()
        hg[i] = pltpu.async_copy(table_hbm.at[idx_all.at[i]], rows[b], gsems[b])

    for i in range(min(NBUF, N_CHUNKS)):
        issue(i)
    for i in range(N_CHUNKS):
        b = i % NBUF
        hg[i].wait()
        hw[i] = pltpu.async_copy(
            rows[b], out_hbm.at[pl.ds(base + i * CHUNK, CHUNK)], wsems[b])
        if i + NBUF < N_CHUNKS:
            issue(i + NBUF)
    for i in range(max(0, N_CHUNKS - NBUF), N_CHUNKS):
        hw[i].wait()


_gather_call = functools.partial(
    pl.kernel,
    out_type=jax.ShapeDtypeStruct((B, D), jnp.float32),
    mesh=plsc.VectorSubcoreMesh(core_axis_name="c", subcore_axis_name="s"),
    scratch_types=(
        [pltpu.VMEM((N_CHUNKS, CHUNK), jnp.int32)]
        + [pltpu.VMEM((CHUNK, D), jnp.float32) for _ in range(NBUF)]
        + [pltpu.SemaphoreType.DMA for _ in range(2 * NBUF)]
    ),
    compiler_params=pltpu.CompilerParams(use_tc_tiling_on_sc=False),
)(_gather_body)


@jax.jit
def kernel(arg0_1, arg1_1):
    idx = arg0_1.reshape(-1).astype(jnp.int32).reshape(NW, N_CHUNKS, CHUNK)
    out = _gather_call(arg1_1, idx)
    return (out.reshape(arg0_1.shape + (D,)),)
